# baseline (speedup 1.0000x reference)
import functools

import jax
import jax.numpy as jnp
from jax import lax
from jax.experimental import pallas as pl
from jax.experimental.pallas import tpu as pltpu
from jax.experimental.pallas import tpu_sc as plsc


def kernel(x, continuous_x_res, tables):
    B, C = x.shape
    _, NCONT, D = continuous_x_res.shape
    V = tables.shape[1]
    OUT_C = C + NCONT
    R = C * D

    tabT = tables.transpose(0, 2, 1).reshape(R, V)
    xT = x.T
    contT = continuous_x_res.transpose(1, 2, 0).reshape(NCONT * D, B)

    NC, NS = 2, 16
    NW = NC * NS
    rpw = R // NW
    crw = (NCONT * D) // NW

    mesh = plsc.VectorSubcoreMesh(core_axis_name="c", subcore_axis_name="s")

    @functools.partial(
        pl.kernel,
        mesh=mesh,
        out_type=jax.ShapeDtypeStruct((OUT_C * D, B), jnp.float32),
        compiler_params=pltpu.CompilerParams(use_tc_tiling_on_sc=False,
                                             needs_layout_passes=False),
        scratch_types=[
            pltpu.VMEM((V,), jnp.float32),
            pltpu.VMEM((B,), jnp.int32),
            pltpu.VMEM((B,), jnp.float32),
            pltpu.SemaphoreType.DMA,
            pltpu.SemaphoreType.DMA,
        ],
    )
    def k(tab_hbm, idx_hbm, cont_hbm, out_hbm, row_v, idx_v, out_v,
          sem_r, sem_c):
        wid = lax.axis_index("s") * NC + lax.axis_index("c")

        cont_cp = pltpu.async_copy(
            cont_hbm.at[pl.ds(wid * crw, crw)],
            out_hbm.at[pl.ds(R + wid * crw, crw)],
            sem_c,
        )

        @pl.loop(0, rpw)
        def _(i):
            r = wid * rpw + i
            c = r // D
            pltpu.sync_copy(idx_hbm.at[c], idx_v)
            pltpu.sync_copy(tab_hbm.at[r], row_v)

            @pl.loop(0, B, step=128)
            def _(b0):
                for u in range(8):
                    idx16 = idx_v[pl.ds(b0 + u * 16, 16)]
                    out_v[pl.ds(b0 + u * 16, 16)] = plsc.load_gather(
                        row_v, [idx16])

            pltpu.sync_copy(out_v, out_hbm.at[r])

        cont_cp.wait()

    out2 = k(tabT, xT, contT)
    return out2.reshape(OUT_C, D, B).transpose(2, 0, 1)
